# baseline (speedup 1.0000x reference)
import jax
import jax.numpy as jnp
from jax.experimental import pallas as pl

_ROWS, _COLS = 1048576, 64
_VROWS, _VCOLS = _ROWS // 2, 128
_BLK = 8192


def _copy_assign_kernel(x_ref, o_ref):
    o_ref[...] = x_ref[...]

    @pl.when(pl.program_id(0) == 0)
    def _():
        o_ref[1:2, 0:64] = jnp.full((1, 64), 9.0, jnp.float32)


def kernel(x):
    xv = x.reshape(_VROWS, _VCOLS)
    out = pl.pallas_call(
        _copy_assign_kernel,
        grid=(_VROWS // _BLK,),
        in_specs=[pl.BlockSpec((_BLK, _VCOLS), lambda i: (i, 0))],
        out_specs=pl.BlockSpec((_BLK, _VCOLS), lambda i: (i, 0)),
        out_shape=jax.ShapeDtypeStruct((_VROWS, _VCOLS), jnp.float32),
    )(xv)
    return out.reshape(_ROWS, _COLS)
